# baseline (speedup 1.0000x reference)
import functools

import jax
import jax.numpy as jnp
from jax.experimental import pallas as pl
from jax.experimental.pallas import tpu as pltpu
from jax.experimental.pallas import tpu_sc as plsc

B = 512
N0, N1 = 20, 10
F = 128
H0, H1 = 64, 32

GRID = 8
C0 = B // GRID
C1 = C0 * N0


NW = 32
R2 = B * N0 * N1
R1 = B * N0
ROWS_OUT_W = R1 // NW
ROWS_IN_W = ROWS_OUT_W * N1
CHUNK_O = 20
CHUNK_I = CHUNK_O * N1
NCH = ROWS_OUT_W // CHUNK_O


def _sc_seg_mean(x2):
    mesh = plsc.VectorSubcoreMesh(core_axis_name="c", subcore_axis_name="s")

    @functools.partial(
        pl.kernel,
        out_type=jax.ShapeDtypeStruct((R1, F), jnp.float32),
        mesh=mesh,
        scratch_types=[
            pltpu.VMEM((CHUNK_I, F), jnp.float32),
            pltpu.VMEM((CHUNK_I, F), jnp.float32),
            pltpu.VMEM((ROWS_OUT_W, F), jnp.float32),
            pltpu.SemaphoreType.DMA,
            pltpu.SemaphoreType.DMA,
        ],
    )
    def k(x2_hbm, out_hbm, buf0, buf1, out_v, sem0, sem1):
        wid = jax.lax.axis_index("s") * 2 + jax.lax.axis_index("c")
        base_in = wid * ROWS_IN_W

        def start(c, buf, sem):
            pltpu.async_copy(
                x2_hbm.at[pl.ds(base_in + c * CHUNK_I, CHUNK_I)], buf, sem)

        def wait(buf, sem):
            pltpu.make_async_copy(
                x2_hbm.at[pl.ds(base_in, CHUNK_I)], buf, sem).wait()

        def compute(c, buf):
            def row(r, carry):
                for f in range(F // 16):
                    sl = pl.ds(f * 16, 16)
                    acc = buf[r * N1, sl]
                    for j in range(1, N1):
                        acc = acc + buf[r * N1 + j, sl]
                    out_v[c * CHUNK_O + r, sl] = acc * (1.0 / N1)
                return carry
            jax.lax.fori_loop(0, CHUNK_O, row, 0)

        start(0, buf0, sem0)

        def pair(g, carry):
            start(2 * g + 1, buf1, sem1)
            wait(buf0, sem0)
            compute(2 * g, buf0)

            @pl.when(g < NCH // 2 - 1)
            def _():
                start(2 * g + 2, buf0, sem0)

            wait(buf1, sem1)
            compute(2 * g + 1, buf1)
            return carry

        jax.lax.fori_loop(0, NCH // 2, pair, 0)
        pltpu.sync_copy(out_v, out_hbm.at[pl.ds(wid * ROWS_OUT_W, ROWS_OUT_W)])

    return k(x2)


def _seg_mean(x, n, inner):
    R, Fdim = x.shape
    b = R // inner
    g = inner // n
    X3 = x.reshape(b, inner, Fdim)
    r_ids = jax.lax.broadcasted_iota(jnp.int32, (b, g, inner), 2)
    s_ids = jax.lax.broadcasted_iota(jnp.int32, (b, g, inner), 1)
    S = jnp.where(r_ids // n == s_ids, 1.0, 0.0).astype(x.dtype)
    out = jax.lax.dot_general(
        S, X3, (((2,), (1,)), ((0,), (0,))),
        preferred_element_type=jnp.float32)
    return out.reshape(R // n, Fdim) * (1.0 / n)


def _group(x0, x1f, a2, W0s, W0n, W1s, W1n):
    h1 = jnp.maximum(
        jnp.dot(x1f, W0s, preferred_element_type=jnp.float32)
        + jnp.dot(a2, W0n, preferred_element_type=jnp.float32), 0.0)
    a1 = _seg_mean(x1f, N0, 160)
    h0 = jnp.maximum(
        jnp.dot(x0, W0s, preferred_element_type=jnp.float32)
        + jnp.dot(a1, W0n, preferred_element_type=jnp.float32), 0.0)
    ah1 = _seg_mean(h1, N0, 160)
    return jnp.maximum(
        jnp.dot(h0, W1s, preferred_element_type=jnp.float32)
        + jnp.dot(ah1, W1n, preferred_element_type=jnp.float32), 0.0)


def _body_raw(x0_r, x1_r, x2_r, w0s_r, w0n_r, w1s_r, w1n_r, o_r):
    W0s = w0s_r[...]
    W0n = w0n_r[...]
    W1s = w1s_r[...]
    W1n = w1n_r[...]
    a2 = _seg_mean(x2_r[...], N1, 640)
    o_r[...] = _group(x0_r[...], x1_r[...], a2, W0s, W0n, W1s, W1n)


def _body_pre(x0n, x1n, a2n, w0s_r, w0n_r, w1s_r, w1n_r, on_r):
    on_r[...] = _group(x0n[...], x1n[...], a2n[...],
                       w0s_r[...], w0n_r[...], w1s_r[...], w1n_r[...])


@jax.jit
def kernel(x_src_0, x_src_1, x_src_2, x_dst_0, x_dst_1, x_dst_2,
           x_neg_0, x_neg_1, x_neg_2, W0_self, W0_neigh, W1_self, W1_neigh):
    x1_spec = pl.BlockSpec((C1, F), lambda i: (i, 0))
    x2_spec = pl.BlockSpec((C1 * N1, F), lambda i: (i, 0))
    x0_spec = pl.BlockSpec((C0, F), lambda i: (i, 0))
    out_spec = pl.BlockSpec((C0, H1), lambda i: (i, 0))
    w_specs = [
        pl.BlockSpec((F, H0), lambda i: (0, 0)),
        pl.BlockSpec((F, H0), lambda i: (0, 0)),
        pl.BlockSpec((H0, H1), lambda i: (0, 0)),
        pl.BlockSpec((H0, H1), lambda i: (0, 0)),
    ]

    a2_neg = _sc_seg_mean(x_neg_2)

    def one_group(x0, x1, x2):
        return pl.pallas_call(
            _body_raw,
            grid=(GRID,),
            in_specs=[x0_spec, x1_spec, x2_spec] + w_specs,
            out_specs=out_spec,
            out_shape=jax.ShapeDtypeStruct((B, H1), jnp.float32),
        )(x0, x1, x2, W0_self, W0_neigh, W1_self, W1_neigh)

    out_sd = (one_group(x_src_0, x_src_1, x_src_2),
              one_group(x_dst_0, x_dst_1, x_dst_2))

    out_n = pl.pallas_call(
        _body_pre,
        grid=(GRID,),
        in_specs=[x0_spec, x1_spec, x1_spec] + w_specs,
        out_specs=out_spec,
        out_shape=jax.ShapeDtypeStruct((B, H1), jnp.float32),
    )(x_neg_0, x_neg_1, a2_neg, W0_self, W0_neigh, W1_self, W1_neigh)

    return (out_sd[0], out_sd[1], out_n)


# speedup vs baseline: 1.3254x; 1.3254x over previous
import functools

import jax
import jax.numpy as jnp
from jax.experimental import pallas as pl

B = 512
N0, N1 = 20, 10
F = 128
H0, H1 = 64, 32

GRID = 8
C0 = B // GRID
C1 = C0 * N0


def _seg_mean(x, n, inner):
    R, Fdim = x.shape
    b = R // inner
    g = inner // n
    X3 = x.reshape(b, inner, Fdim)
    r_ids = jax.lax.broadcasted_iota(jnp.int32, (b, g, inner), 2)
    s_ids = jax.lax.broadcasted_iota(jnp.int32, (b, g, inner), 1)
    S = jnp.where(r_ids // n == s_ids, 1.0, 0.0).astype(x.dtype)
    out = jax.lax.dot_general(
        S, X3, (((2,), (1,)), ((0,), (0,))),
        preferred_element_type=jnp.float32)
    return out.reshape(R // n, Fdim) * (1.0 / n)


def _body(x0s, x1s, x2s, x0d, x1d, x2d, x0n, x1n, x2n,
          w0s_r, w0n_r, w1s_r, w1n_r, os_r, od_r, on_r):
    W0s = w0s_r[...]
    W0n = w0n_r[...]
    W1s = w1s_r[...]
    W1n = w1n_r[...]
    for x0_r, x1_r, x2_r, o_r in ((x0s, x1s, x2s, os_r),
                                  (x0d, x1d, x2d, od_r),
                                  (x0n, x1n, x2n, on_r)):
        x1f = x1_r[...]
        a2 = _seg_mean(x2_r[...], N1, 640)
        h1 = jnp.maximum(
            jnp.dot(x1f, W0s, preferred_element_type=jnp.float32)
            + jnp.dot(a2, W0n, preferred_element_type=jnp.float32), 0.0)
        a1 = _seg_mean(x1f, N0, 160)
        h0 = jnp.maximum(
            jnp.dot(x0_r[...], W0s, preferred_element_type=jnp.float32)
            + jnp.dot(a1, W0n, preferred_element_type=jnp.float32), 0.0)
        ah1 = _seg_mean(h1, N0, 160)
        o_r[...] = jnp.maximum(
            jnp.dot(h0, W1s, preferred_element_type=jnp.float32)
            + jnp.dot(ah1, W1n, preferred_element_type=jnp.float32), 0.0)


@jax.jit
def kernel(x_src_0, x_src_1, x_src_2, x_dst_0, x_dst_1, x_dst_2,
           x_neg_0, x_neg_1, x_neg_2, W0_self, W0_neigh, W1_self, W1_neigh):
    x1_specs = pl.BlockSpec((C1, F), lambda i: (i, 0))
    x2_specs = pl.BlockSpec((C1 * N1, F), lambda i: (i, 0))
    x0_specs = pl.BlockSpec((C0, F), lambda i: (i, 0))
    out_spec = pl.BlockSpec((C0, H1), lambda i: (i, 0))

    def r1(x):
        return x

    def r2(x):
        return x

    in_specs = [x0_specs, x1_specs, x2_specs] * 3 + [
        pl.BlockSpec((F, H0), lambda i: (0, 0)),
        pl.BlockSpec((F, H0), lambda i: (0, 0)),
        pl.BlockSpec((H0, H1), lambda i: (0, 0)),
        pl.BlockSpec((H0, H1), lambda i: (0, 0)),
    ]
    out_shape = [jax.ShapeDtypeStruct((B, H1), jnp.float32)] * 3
    out_specs = [out_spec] * 3

    return tuple(pl.pallas_call(
        _body,
        grid=(GRID,),
        in_specs=in_specs,
        out_specs=out_specs,
        out_shape=out_shape,
    )(x_src_0, r1(x_src_1), r2(x_src_2),
      x_dst_0, r1(x_dst_1), r2(x_dst_2),
      x_neg_0, r1(x_neg_1), r2(x_neg_2),
      W0_self, W0_neigh, W1_self, W1_neigh))
